# baseline (speedup 1.0000x reference)
import functools

import jax
import jax.numpy as jnp
from jax import lax
from jax.experimental import pallas as pl
from jax.experimental.pallas import tpu as pltpu
from jax.experimental.pallas import tpu_sc as plsc

EPS = 1e-5

_NC = 2
_NS = 16
_NW = _NC * _NS

_EB = 96


def _sc_pad(n):
  chunk = _NS * _EB
  return (n + chunk - 1) // chunk * chunk


@functools.lru_cache(maxsize=None)
def _make_sc_aggregate(n, nb, h, mode):
  assert nb % 3 == 0 and nb >= 9, "ring-3 pipeline assumes nb % 3 == 0"
  ntrip = nb // 3 - 1
  n_pad = _sc_pad(n)
  rpt = n_pad // _NS
  nz = rpt // _EB

  mesh = plsc.VectorSubcoreMesh(core_axis_name="c", subcore_axis_name="s")

  out_type = [jax.ShapeDtypeStruct((_NC, n_pad, h), jnp.float32)]
  if mode == "sum":
    scratch = [
        pltpu.VMEM((2, _EB), jnp.int32), pltpu.VMEM((2, _EB), jnp.int32),
        pltpu.VMEM((2, _EB), jnp.int32),
        pltpu.VMEM((_EB, h), jnp.float32), pltpu.VMEM((_EB, h), jnp.float32),
        pltpu.VMEM((_EB, h), jnp.float32),
        pltpu.VMEM_SHARED((n_pad, h), jnp.float32),
        pltpu.SemaphoreType.DMA, pltpu.SemaphoreType.DMA,
        pltpu.SemaphoreType.DMA, pltpu.SemaphoreType.DMA,
        pltpu.SemaphoreType.DMA, pltpu.SemaphoreType.DMA,
        pltpu.SemaphoreType.DMA, pltpu.SemaphoreType.DMA,
        pltpu.SemaphoreType.DMA,
    ]
  else:
    scratch = [
        pltpu.VMEM((2, _EB), jnp.int32),
        pltpu.VMEM((_EB, h), jnp.float32),
        pltpu.VMEM_SHARED((n_pad, h), jnp.float32),
    ]

  def body(*refs):
    if mode == "sum":
      (x_hbm, sd_hbm, sums_hbm,
       idx0, idx1, idx2, rows0, rows1, rows2, acc,
       is0, is1, is2, gs0, gs1, gs2, cs0, cs1, cs2) = refs
      idx = (idx0, idx1, idx2)
      rows = (rows0, rows1, rows2)
      isem = (is0, is1, is2)
      gsem = (gs0, gs1, gs2)
      csem = (cs0, cs1, cs2)
    else:
      sd_hbm, sums_hbm, idx0, rows0, acc = refs
    c = lax.axis_index("c")
    s = lax.axis_index("s")
    wid = c * _NS + s
    base = wid * nb

    zero16 = jnp.zeros((16,), jnp.float32)
    row0 = s * rpt

    def zero_acc(buf):
      def fill_zero(i, _):
        for l in range(h // 16):
          buf[i, pl.ds(l * 16, 16)] = zero16
        return 0
      lax.fori_loop(0, _EB, fill_zero, 0)
      for k in range(nz):
        pltpu.sync_copy(buf, acc.at[pl.ds(row0 + k * _EB, _EB)])

    if mode == "sum":
      def idx_start(j, b):
        pltpu.async_copy(sd_hbm.at[base + j], idx[b], isem[b])

      def idx_wait(b):
        pltpu.make_async_copy(sd_hbm.at[0], idx[b], isem[b]).wait()

      def gather_start(b):
        pltpu.async_copy(x_hbm.at[idx[b].at[0]], rows[b], gsem[b])

      def gather_wait(b):
        pltpu.make_async_copy(x_hbm.at[idx[b].at[0]], rows[b],
                              gsem[b]).wait()

      def scatter_start(b):
        pltpu.async_copy(rows[b], acc.at[idx[b].at[1]], csem[b], add=True)

      def scatter_wait(b):
        pltpu.make_async_copy(rows[b], acc.at[idx[b].at[1]],
                              csem[b]).wait()

      idx_start(0, 0)
      idx_start(1, 1)
      idx_wait(0)
      gather_start(0)
      zero_acc(rows2)
      plsc.subcore_barrier()
      gather_wait(0)
      scatter_start(0)
      idx_start(2, 2)
      idx_wait(1)
      gather_start(1)

      def triple(q, _):
        j0 = 3 * q + 1
        for dj in range(3):
          j = j0 + dj
          b = (1 + dj) % 3
          b1 = (2 + dj) % 3
          b2 = dj % 3
          gather_wait(b)
          scatter_start(b)
          scatter_wait(b2)
          idx_start(j + 2, b2)
          idx_wait(b1)
          gather_start(b1)
        return 0
      lax.fori_loop(0, ntrip, triple, 0)
      gather_wait(1)
      scatter_start(1)
      idx_wait(2)
      gather_start(2)
      gather_wait(2)
      scatter_start(2)
      scatter_wait(0)
      scatter_wait(1)
      scatter_wait(2)
    else:
      zero_acc(rows0)
      plsc.subcore_barrier()

      def fill_ones(i, _):
        for l in range(h // 16):
          rows0[i, pl.ds(l * 16, 16)] = zero16 + 1.0
        return 0
      lax.fori_loop(0, _EB, fill_ones, 0)

      def step(j, _):
        pltpu.sync_copy(sd_hbm.at[base + j], idx0)
        pltpu.sync_copy(rows0, acc.at[idx0.at[1]], add=True)
        return 0
      lax.fori_loop(0, nb, step, 0)
    plsc.subcore_barrier()

    for k in range(nz):
      r = row0 + k * _EB
      pltpu.sync_copy(acc.at[pl.ds(r, _EB)], sums_hbm.at[c, pl.ds(r, _EB)])

  return pl.kernel(body, out_type=out_type, mesh=mesh, scratch_types=scratch)


def _rep(shape):
  return pl.BlockSpec(shape, lambda i: (0,) * len(shape))


@functools.lru_cache(maxsize=None)
def _make_tc_layer(n, h, ho, bn, first, head):
  assert n % bn == 0
  grid = (n // bn,)

  def body(*refs):
    it = iter(refs)
    sums = next(it)
    cnt = next(it)
    hin = next(it)
    wl = next(it)
    bli = next(it)
    wr = next(it)
    g = next(it)
    b = next(it)
    rm = next(it)
    rv = next(it)
    if head:
      lng = next(it)
      lnb = next(it)
      w1 = next(it)
      b1 = next(it)
      g2 = next(it)
      b2 = next(it)
      rm2 = next(it)
      rv2 = next(it)
      w2 = next(it)
      b2o = next(it)
    out = next(it)
    if first:
      inv_out = next(it)

    ssum = sums[0] + sums[1]
    if first:
      inv = 1.0 / jnp.maximum(cnt[0, :, 0:1] + cnt[1, :, 0:1], 1.0)
      inv_out[:] = inv
    else:
      inv = cnt[:]
    mean = ssum * inv
    z = (jnp.dot(mean, wl[:], preferred_element_type=jnp.float32)
         + bli[:]
         + jnp.dot(hin[:], wr[:], preferred_element_type=jnp.float32))
    scale = g[:] / jnp.sqrt(rv[:] + EPS)
    z = (z - rm[:]) * scale + b[:]
    z = jnp.maximum(z, 0.0)
    if first:
      z = z + hin[:]
    if not head:
      out[:] = z
      return
    mu = jnp.mean(z, axis=-1, keepdims=True)
    var = jnp.mean(jnp.square(z - mu), axis=-1, keepdims=True)
    z = (z - mu) * lax.rsqrt(var + EPS) * lng[:] + lnb[:]
    t = jnp.dot(z, w1[:], preferred_element_type=jnp.float32) + b1[:]
    scale2 = g2[:] / jnp.sqrt(rv2[:] + EPS)
    t = jnp.maximum((t - rm2[:]) * scale2 + b2[:], 0.0)
    out[:] = jnp.sum(t * w2[:], axis=-1, keepdims=True) + b2o[:]

  in_specs = [
      pl.BlockSpec((2, bn, h), lambda i: (0, i, 0)),
      (pl.BlockSpec((2, bn, h), lambda i: (0, i, 0)) if first
       else pl.BlockSpec((bn, 1), lambda i: (i, 0))),
      pl.BlockSpec((bn, h), lambda i: (i, 0)),
      _rep((h, h)), _rep((1, h)), _rep((h, h)),
      _rep((1, h)), _rep((1, h)), _rep((1, h)), _rep((1, h)),
  ]
  if head:
    in_specs += [
        _rep((1, h)), _rep((1, h)),
        _rep((h, ho)), _rep((1, ho)),
        _rep((1, ho)), _rep((1, ho)), _rep((1, ho)), _rep((1, ho)),
        _rep((1, ho)), _rep((1, 1)),
    ]
    out_spec = pl.BlockSpec((bn, 1), lambda i: (i, 0))
    out_shape = jax.ShapeDtypeStruct((n, 1), jnp.float32)
  else:
    out_spec = pl.BlockSpec((bn, h), lambda i: (i, 0))
    out_shape = jax.ShapeDtypeStruct((n, h), jnp.float32)
  if first:
    out_spec = [out_spec, pl.BlockSpec((bn, 1), lambda i: (i, 0))]
    out_shape = [out_shape, jax.ShapeDtypeStruct((n, 1), jnp.float32)]

  return pl.pallas_call(
      body, grid=grid, in_specs=in_specs, out_specs=out_spec,
      out_shape=out_shape)


def kernel(x, edge_index, Wl, bl, Wr, bn_g, bn_b, bn_rm, bn_rv, ln_g, ln_b,
           lin1_W, lin1_b, bno_g, bno_b, bno_rm, bno_rv, lin2_W, lin2_b):
  n, h = x.shape
  e = edge_index.shape[1]
  L = Wl.shape[0]
  ho = lin1_W.shape[1]

  assert e % _NW == 0
  epw = e // _NW
  nb = -(-epw // _EB)
  nb = -(-nb // 3) * 3
  pad = nb * _EB - epw
  n_pad = _sc_pad(n)
  sr = edge_index[0].reshape(_NW, epw)
  dr = edge_index[1].reshape(_NW, epw)
  if pad:
    pi = jnp.arange(pad, dtype=jnp.int32)[None, :]
    wi = jnp.arange(_NW, dtype=jnp.int32)[:, None]
    sr = jnp.concatenate([sr, (pi * 89 + wi * 997) % n], axis=1)
    dr = jnp.concatenate([dr, n + (pi + wi * 7) % (n_pad - n)], axis=1)
  sd = jnp.stack([sr.reshape(_NW, nb, _EB), dr.reshape(_NW, nb, _EB)],
                 axis=2).reshape(_NW * nb, 2, _EB)

  agg = _make_sc_aggregate(n, nb, h, "sum")
  agg_cnt = _make_sc_aggregate(n, nb, h, "cnt")
  bn = 400

  def r2(v):
    return v.reshape(1, -1)

  hcur = x
  (cnt_parts,) = agg_cnt(sd)
  inv = None
  for i in range(L):
    (sums_parts,) = agg(hcur, sd)
    first = i == 0
    head = i == L - 1
    layer_fn = _make_tc_layer(n, h, ho, bn, first, head)
    args = [sums_parts, cnt_parts if first else inv, hcur,
            Wl[i], r2(bl[i]), Wr[i],
            r2(bn_g[i]), r2(bn_b[i]), r2(bn_rm[i]), r2(bn_rv[i])]
    if head:
      args += [r2(ln_g), r2(ln_b), lin1_W, r2(lin1_b),
               r2(bno_g), r2(bno_b), r2(bno_rm), r2(bno_rv),
               lin2_W.reshape(1, -1), lin2_b.reshape(1, 1)]
    if first:
      hcur, inv = layer_fn(*args)
    else:
      hcur = layer_fn(*args)
  return hcur[:, 0]


# speedup vs baseline: 1.1378x; 1.1378x over previous
import functools

import jax
import jax.numpy as jnp
from jax import lax
from jax.experimental import pallas as pl
from jax.experimental.pallas import tpu as pltpu
from jax.experimental.pallas import tpu_sc as plsc

EPS = 1e-5

_NC = 2
_NS = 16
_NW = _NC * _NS

_EB = 128


def _sc_pad(n):
  chunk = _NS * _EB
  return (n + chunk - 1) // chunk * chunk


@functools.lru_cache(maxsize=None)
def _make_sc_aggregate(n, nb, h, mode):
  assert nb % 2 == 1 and nb >= 3, "pipeline assumes an odd batch count"
  npairs = (nb - 1) // 2
  n_pad = _sc_pad(n)
  rpt = n_pad // _NS
  nz = rpt // _EB
  hc = h if mode == "sum" else 64

  mesh = plsc.VectorSubcoreMesh(core_axis_name="c", subcore_axis_name="s")

  out_type = [jax.ShapeDtypeStruct((_NC, n_pad, hc), jnp.float32)]
  if mode == "sum":
    scratch = [
        pltpu.VMEM((2, _EB), jnp.int32), pltpu.VMEM((2, _EB), jnp.int32),
        pltpu.VMEM((_EB, h), jnp.float32), pltpu.VMEM((_EB, h), jnp.float32),
        pltpu.VMEM_SHARED((n_pad, h), jnp.float32),
        pltpu.SemaphoreType.DMA, pltpu.SemaphoreType.DMA,
        pltpu.SemaphoreType.DMA, pltpu.SemaphoreType.DMA,
    ]
  else:
    scratch = [
        pltpu.VMEM((2, _EB), jnp.int32),
        pltpu.VMEM((_EB, hc), jnp.float32),
        pltpu.VMEM_SHARED((n_pad, hc), jnp.float32),
    ]

  def body(*refs):
    if mode == "sum":
      (x_hbm, sd_hbm, sums_hbm,
       idx0, idx1, rows0, rows1, acc, gs0, gs1, is0, is1) = refs
      idx = (idx0, idx1)
      rows = (rows0, rows1)
      gsem = (gs0, gs1)
      isem = (is0, is1)
    else:
      sd_hbm, sums_hbm, idx0, rows0, acc = refs
    c = lax.axis_index("c")
    s = lax.axis_index("s")
    wid = c * _NS + s
    base = wid * nb

    zero16 = jnp.zeros((16,), jnp.float32)
    row0 = s * rpt

    def zero_acc(buf):
      def fill_zero(i, _):
        for l in range(hc // 16):
          buf[i, pl.ds(l * 16, 16)] = zero16
        return 0
      lax.fori_loop(0, _EB, fill_zero, 0)
      for k in range(nz):
        pltpu.sync_copy(buf, acc.at[pl.ds(row0 + k * _EB, _EB)])

    if mode == "sum":
      def idx_start(j, b):
        pltpu.async_copy(sd_hbm.at[base + j], idx[b], isem[b])

      def idx_wait(b):
        pltpu.make_async_copy(sd_hbm.at[0], idx[b], isem[b]).wait()

      def gather_start(b):
        pltpu.async_copy(x_hbm.at[idx[b].at[0]], rows[b], gsem[b])

      def gather_wait(b):
        pltpu.make_async_copy(x_hbm.at[idx[b].at[0]], rows[b],
                              gsem[b]).wait()

      def scatter(b):
        pltpu.sync_copy(rows[b], acc.at[idx[b].at[1]], add=True)

      idx_start(0, 0)
      idx_wait(0)
      gather_start(0)
      idx_start(1, 1)
      zero_acc(rows1)
      plsc.subcore_barrier()

      def pair(p, _):
        j0 = 2 * p
        idx_wait(1)
        gather_wait(0)
        gather_start(1)
        scatter(0)
        idx_start(j0 + 2, 0)
        gather_wait(1)
        idx_wait(0)
        gather_start(0)
        scatter(1)

        @pl.when(p < npairs - 1)
        def _():
          idx_start(j0 + 3, 1)
        return 0
      lax.fori_loop(0, npairs, pair, 0)
      gather_wait(0)
      scatter(0)
    else:
      zero_acc(rows0)
      plsc.subcore_barrier()

      def fill_ones(i, _):
        for l in range(hc // 16):
          rows0[i, pl.ds(l * 16, 16)] = zero16 + 1.0
        return 0
      lax.fori_loop(0, _EB, fill_ones, 0)

      def step(j, _):
        pltpu.sync_copy(sd_hbm.at[base + j], idx0)
        pltpu.sync_copy(rows0, acc.at[idx0.at[1]], add=True)
        return 0
      lax.fori_loop(0, nb, step, 0)
    plsc.subcore_barrier()

    for k in range(nz):
      r = row0 + k * _EB
      pltpu.sync_copy(acc.at[pl.ds(r, _EB)], sums_hbm.at[c, pl.ds(r, _EB)])

  return pl.kernel(body, out_type=out_type, mesh=mesh, scratch_types=scratch)


def _rep(shape):
  return pl.BlockSpec(shape, lambda i: (0,) * len(shape))


@functools.lru_cache(maxsize=None)
def _make_tc_layer(n, h, ho, bn, first, head):
  assert n % bn == 0
  grid = (n // bn,)

  def body(*refs):
    it = iter(refs)
    sums = next(it)
    cnt = next(it)
    hin = next(it)
    wl = next(it)
    bli = next(it)
    wr = next(it)
    g = next(it)
    b = next(it)
    rm = next(it)
    rv = next(it)
    if head:
      lng = next(it)
      lnb = next(it)
      w1 = next(it)
      b1 = next(it)
      g2 = next(it)
      b2 = next(it)
      rm2 = next(it)
      rv2 = next(it)
      w2 = next(it)
      b2o = next(it)
    out = next(it)
    if first:
      inv_out = next(it)

    ssum = sums[0] + sums[1]
    if first:
      inv = 1.0 / jnp.maximum(cnt[0, :, 0:1] + cnt[1, :, 0:1], 1.0)
      inv_out[:] = inv
    else:
      inv = cnt[:]
    mean = ssum * inv
    z = (jnp.dot(mean, wl[:], preferred_element_type=jnp.float32)
         + bli[:]
         + jnp.dot(hin[:], wr[:], preferred_element_type=jnp.float32))
    scale = g[:] / jnp.sqrt(rv[:] + EPS)
    z = (z - rm[:]) * scale + b[:]
    z = jnp.maximum(z, 0.0)
    if first:
      z = z + hin[:]
    if not head:
      out[:] = z
      return
    mu = jnp.mean(z, axis=-1, keepdims=True)
    var = jnp.mean(jnp.square(z - mu), axis=-1, keepdims=True)
    z = (z - mu) * lax.rsqrt(var + EPS) * lng[:] + lnb[:]
    t = jnp.dot(z, w1[:], preferred_element_type=jnp.float32) + b1[:]
    scale2 = g2[:] / jnp.sqrt(rv2[:] + EPS)
    t = jnp.maximum((t - rm2[:]) * scale2 + b2[:], 0.0)
    out[:] = jnp.sum(t * w2[:], axis=-1, keepdims=True) + b2o[:]

  in_specs = [
      pl.BlockSpec((2, bn, h), lambda i: (0, i, 0)),
      (pl.BlockSpec((2, bn, 64), lambda i: (0, i, 0)) if first
       else pl.BlockSpec((bn, 1), lambda i: (i, 0))),
      pl.BlockSpec((bn, h), lambda i: (i, 0)),
      _rep((h, h)), _rep((1, h)), _rep((h, h)),
      _rep((1, h)), _rep((1, h)), _rep((1, h)), _rep((1, h)),
  ]
  if head:
    in_specs += [
        _rep((1, h)), _rep((1, h)),
        _rep((h, ho)), _rep((1, ho)),
        _rep((1, ho)), _rep((1, ho)), _rep((1, ho)), _rep((1, ho)),
        _rep((1, ho)), _rep((1, 1)),
    ]
    out_spec = pl.BlockSpec((bn, 1), lambda i: (i, 0))
    out_shape = jax.ShapeDtypeStruct((n, 1), jnp.float32)
  else:
    out_spec = pl.BlockSpec((bn, h), lambda i: (i, 0))
    out_shape = jax.ShapeDtypeStruct((n, h), jnp.float32)
  if first:
    out_spec = [out_spec, pl.BlockSpec((bn, 1), lambda i: (i, 0))]
    out_shape = [out_shape, jax.ShapeDtypeStruct((n, 1), jnp.float32)]

  return pl.pallas_call(
      body, grid=grid, in_specs=in_specs, out_specs=out_spec,
      out_shape=out_shape)


def kernel(x, edge_index, Wl, bl, Wr, bn_g, bn_b, bn_rm, bn_rv, ln_g, ln_b,
           lin1_W, lin1_b, bno_g, bno_b, bno_rm, bno_rv, lin2_W, lin2_b):
  n, h = x.shape
  e = edge_index.shape[1]
  L = Wl.shape[0]
  ho = lin1_W.shape[1]

  assert e % _NW == 0
  epw = e // _NW
  nb = -(-epw // _EB)
  pad = nb * _EB - epw
  n_pad = _sc_pad(n)
  sr = edge_index[0].reshape(_NW, epw)
  dr = edge_index[1].reshape(_NW, epw)
  if pad:
    pi = jnp.arange(pad, dtype=jnp.int32)[None, :]
    wi = jnp.arange(_NW, dtype=jnp.int32)[:, None]
    sr = jnp.concatenate([sr, (pi * 89 + wi * 997) % n], axis=1)
    dr = jnp.concatenate([dr, n + (pi + wi * 7) % (n_pad - n)], axis=1)
  sd = jnp.stack([sr.reshape(_NW, nb, _EB), dr.reshape(_NW, nb, _EB)],
                 axis=2).reshape(_NW * nb, 2, _EB)

  agg = _make_sc_aggregate(n, nb, h, "sum")
  agg_cnt = _make_sc_aggregate(n, nb, h, "cnt")
  bn = 400

  def r2(v):
    return v.reshape(1, -1)

  hcur = x
  (cnt_parts,) = agg_cnt(sd)
  inv = None
  for i in range(L):
    (sums_parts,) = agg(hcur, sd)
    first = i == 0
    head = i == L - 1
    layer_fn = _make_tc_layer(n, h, ho, bn, first, head)
    args = [sums_parts, cnt_parts if first else inv, hcur,
            Wl[i], r2(bl[i]), Wr[i],
            r2(bn_g[i]), r2(bn_b[i]), r2(bn_rm[i]), r2(bn_rv[i])]
    if head:
      args += [r2(ln_g), r2(ln_b), lin1_W, r2(lin1_b),
               r2(bno_g), r2(bno_b), r2(bno_rm), r2(bno_rv),
               lin2_W.reshape(1, -1), lin2_b.reshape(1, 1)]
    if first:
      hcur, inv = layer_fn(*args)
    else:
      hcur = layer_fn(*args)
  return hcur[:, 0]
